# Initial kernel scaffold; baseline (speedup 1.0000x reference)
import jax
import jax.numpy as jnp
from jax.experimental import pallas as pl


def kernel(x, w_router, w1, w2, w_out):
    raise NotImplementedError("write your pallas kernel here")


# speedup vs baseline: 1.1183x; 1.1183x over previous
import functools

import jax
import jax.numpy as jnp
from jax import lax
from jax.experimental import pallas as pl
from jax.experimental.pallas import tpu as pltpu
from jax.experimental.pallas import tpu_sc as plsc

T = 2048
D = 768
E = 8
K = 2
CAP = 640
DFF = 3072
ROWS = E * CAP
TRASH = ROWS
TAB = 5136
NFF = 4
FFC = DFF // NFF
XPAD = T + 8
CH = 256
NCH = T // CH

NC, NS = 2, 16
NW = NC * NS
RPW = ROWS // NW
HRPW = RPW // 2
TPW = T // NW
HTOK = TPW // 2
LN = D // 16


def _route_body(x_ref, wr_ref, noise_ref, key_ref, g_ref, fpair_ref,
                a_scr, cums_scr):
    x = x_ref[...]
    logits = lax.dot_general(
        x, wr_ref[...], (((1,), (0,)), ((), ())),
        preferred_element_type=jnp.float32)
    z = logits + noise_ref[...]
    m = jnp.max(z, axis=1, keepdims=True)
    p = jnp.exp(z - m)
    probs = p / jnp.sum(p, axis=1, keepdims=True)

    lane = lax.broadcasted_iota(jnp.int32, (T, E), 1)
    g0 = jnp.max(probs, axis=1, keepdims=True)
    i0 = jnp.min(jnp.where(probs == g0, lane, E), axis=1, keepdims=True)
    probs1 = jnp.where(lane == i0, -jnp.inf, probs)
    g1 = jnp.max(probs1, axis=1, keepdims=True)
    i1 = jnp.min(jnp.where(probs1 == g1, lane, E), axis=1, keepdims=True)

    oh0 = (lane == i0).astype(jnp.float32)
    oh1 = (lane == i1).astype(jnp.float32)
    a_scr[...] = oh0 + oh1

    r = lax.broadcasted_iota(jnp.int32, (CH, CH), 0)
    c = lax.broadcasted_iota(jnp.int32, (CH, CH), 1)
    lstrict = (r > c).astype(jnp.float32)

    def body(i, carry):
        chunk = a_scr[pl.ds(i * CH, CH), :]
        cums_scr[pl.ds(i * CH, CH), :] = lax.dot_general(
            lstrict, chunk, (((1,), (0,)), ((), ())),
            preferred_element_type=jnp.float32) + carry
        return carry + jnp.sum(chunk, axis=0, keepdims=True)

    counts = lax.fori_loop(0, NCH, body, jnp.zeros((1, E), jnp.float32))

    cums = cums_scr[...]
    pos0 = jnp.sum(cums * oh0, axis=1, keepdims=True).astype(jnp.int32)
    pos1 = jnp.sum(cums * oh1, axis=1, keepdims=True).astype(jnp.int32)
    keep0 = pos0 < CAP
    keep1 = pos1 < CAP
    key0 = jnp.where(keep0, i0 * CAP + pos0, TRASH)
    key1 = jnp.where(keep1, i1 * CAP + pos1, TRASH)

    cmin = jnp.min(counts)
    cl = lax.broadcasted_iota(jnp.int32, (1, E), 1)
    emin = jnp.min(jnp.where(counts == cmin, cl, E))
    zrow = emin * CAP + cmin.astype(jnp.int32)

    key_ref[...] = jnp.concatenate([key0, key1], axis=1)
    g_ref[...] = jnp.concatenate([g0, g1], axis=1)
    fpair_ref[...] = jnp.concatenate(
        [jnp.where(keep0, key0, zrow), jnp.where(keep1, key1, zrow)], axis=1)


_route = pl.pallas_call(
    _route_body,
    out_shape=(
        jax.ShapeDtypeStruct((T, K), jnp.int32),
        jax.ShapeDtypeStruct((T, K), jnp.float32),
        jax.ShapeDtypeStruct((T, K), jnp.int32),
    ),
    scratch_shapes=[
        pltpu.VMEM((T, E), jnp.float32),
        pltpu.VMEM((T, E), jnp.float32),
    ],
)


def _dispatch_body(x_hbm, key_hbm, g_hbm, buf_hbm, gate_hbm,
                   key_v, g_v, tsrc_v, gate_v, rows_v, sem):
    wid = lax.axis_index("s") * NC + lax.axis_index("c")
    pltpu.sync_copy(key_hbm, key_v)
    pltpu.sync_copy(g_hbm, g_v)

    fill16 = jnp.full((16,), T, jnp.int32)
    zero16 = jnp.zeros((16,), jnp.float32)

    def initb(i, _):
        tsrc_v[pl.ds(i * 16, 16)] = fill16
        gate_v[pl.ds(i * 16, 16)] = zero16
        return 0

    lax.fori_loop(0, TAB // 16, initb, 0)

    half_iota = jnp.right_shift(lax.iota(jnp.int32, 16), 1)

    def scat(i, _):
        k16 = key_v[pl.ds(i * 16, 16)]
        g16 = g_v[pl.ds(i * 16, 16)]
        t16 = i * 8 + half_iota
        plsc.store_scatter(tsrc_v, [k16], t16)
        plsc.store_scatter(gate_v, [k16], g16)
        return 0

    lax.fori_loop(0, (T * K) // 16, scat, 0)

    base = wid * RPW
    for h in range(2):
        pltpu.async_copy(
            x_hbm.at[tsrc_v.at[pl.ds(base + h * HRPW, HRPW)]],
            rows_v, sem).wait()
        pltpu.sync_copy(rows_v, buf_hbm.at[pl.ds(base + h * HRPW, HRPW)])
    pltpu.sync_copy(gate_v.at[pl.ds(base, RPW)], gate_hbm.at[pl.ds(base, RPW)])


@functools.cache
def _dispatch():
    return functools.partial(
        pl.kernel,
        out_type=(
            jax.ShapeDtypeStruct((ROWS, D), jnp.float32),
            jax.ShapeDtypeStruct((ROWS,), jnp.float32),
        ),
        mesh=plsc.VectorSubcoreMesh(
            core_axis_name="c", subcore_axis_name="s",
            num_cores=NC, num_subcores=NS),
        scratch_types=[
            pltpu.VMEM((T * K,), jnp.int32),
            pltpu.VMEM((T * K,), jnp.float32),
            pltpu.VMEM((TAB,), jnp.int32),
            pltpu.VMEM((TAB,), jnp.float32),
            pltpu.VMEM((HRPW, D), jnp.float32),
            pltpu.SemaphoreType.DMA,
        ],
        compiler_params=pltpu.CompilerParams(needs_layout_passes=False),
    )(_dispatch_body)


def _ffn_body(buf_ref, gate_ref, w1_ref, w2_ref, wo_ref, out_ref):
    j = pl.program_id(1)
    xb = buf_ref[...].astype(jnp.bfloat16)
    h1 = lax.dot_general(xb, w1_ref[0], (((1,), (0,)), ((), ())),
                         preferred_element_type=jnp.float32)
    h2 = lax.dot_general(xb, w2_ref[0], (((1,), (0,)), ((), ())),
                         preferred_element_type=jnp.float32)
    h = (h1 * jax.nn.sigmoid(h1) * h2).astype(jnp.bfloat16)
    part = lax.dot_general(h, wo_ref[0], (((1,), (0,)), ((), ())),
                           preferred_element_type=jnp.float32)

    @pl.when(j == 0)
    def _():
        out_ref[...] = part

    @pl.when(j != 0)
    def _():
        out_ref[...] = out_ref[...] + part

    @pl.when(j == NFF - 1)
    def _():
        out_ref[...] = out_ref[...] * gate_ref[...]


_ffn = pl.pallas_call(
    _ffn_body,
    grid=(E, NFF),
    in_specs=[
        pl.BlockSpec((CAP, D), lambda e, j: (e, 0)),
        pl.BlockSpec((CAP, 1), lambda e, j: (e, 0)),
        pl.BlockSpec((1, D, FFC), lambda e, j: (e, 0, j)),
        pl.BlockSpec((1, D, FFC), lambda e, j: (e, 0, j)),
        pl.BlockSpec((1, FFC, D), lambda e, j: (e, j, 0)),
    ],
    out_specs=pl.BlockSpec((CAP, D), lambda e, j: (e, 0)),
    out_shape=jax.ShapeDtypeStruct((ROWS, D), jnp.float32),
    compiler_params=pltpu.CompilerParams(
        dimension_semantics=("arbitrary", "arbitrary")),
)


def _combine_body(outbuf_hbm, fpair_hbm, out_hbm, fp_v, rows_v, acc_v, sem):
    wid = lax.axis_index("s") * NC + lax.axis_index("c")
    pltpu.sync_copy(fpair_hbm.at[pl.ds(wid * TPW * K, TPW * K)], fp_v)
    for h in range(2):
        pltpu.async_copy(
            outbuf_hbm.at[fp_v.at[pl.ds(h * HTOK * K, HTOK * K)]],
            rows_v, sem).wait()

        def tokb(tk, _):
            def chb(c, _):
                a = rows_v[2 * tk, pl.ds(c * 16, 16)]
                b = rows_v[2 * tk + 1, pl.ds(c * 16, 16)]
                acc_v[tk, pl.ds(c * 16, 16)] = a + b
                return 0
            lax.fori_loop(0, LN, chb, 0)
            return 0

        lax.fori_loop(0, HTOK, tokb, 0)
        pltpu.sync_copy(acc_v, out_hbm.at[pl.ds(wid * TPW + h * HTOK, HTOK)])


@functools.cache
def _combine():
    return functools.partial(
        pl.kernel,
        out_type=jax.ShapeDtypeStruct((T, D), jnp.float32),
        mesh=plsc.VectorSubcoreMesh(
            core_axis_name="c", subcore_axis_name="s",
            num_cores=NC, num_subcores=NS),
        scratch_types=[
            pltpu.VMEM((TPW * K,), jnp.int32),
            pltpu.VMEM((HTOK * K, D), jnp.float32),
            pltpu.VMEM((HTOK, D), jnp.float32),
            pltpu.SemaphoreType.DMA,
        ],
        compiler_params=pltpu.CompilerParams(needs_layout_passes=False),
    )(_combine_body)


def kernel(x, w_router, w1, w2, w_out):
    xt = x.reshape(T, D).astype(jnp.float32)
    noise = jax.random.gumbel(
        jax.random.key(42), (T, E), dtype=jnp.float32) * 0.05

    key2, g2, fpair2 = _route(xt, w_router, noise)
    x_pad = jnp.concatenate(
        [xt, jnp.zeros((XPAD - T, D), jnp.float32)], axis=0)
    buf, gate_tab = _dispatch()(x_pad, key2.reshape(T * K), g2.reshape(T * K))
    out_buf = _ffn(buf, gate_tab.reshape(ROWS, 1),
                   w1.astype(jnp.bfloat16), w2.astype(jnp.bfloat16),
                   w_out.astype(jnp.bfloat16))
    out = _combine()(out_buf, fpair2.reshape(T * K))
    return out.reshape(1, T, D)


# speedup vs baseline: 1.5845x; 1.4168x over previous
import functools

import jax
import jax.numpy as jnp
from jax import lax
from jax.experimental import pallas as pl
from jax.experimental.pallas import tpu as pltpu
from jax.experimental.pallas import tpu_sc as plsc

T = 2048
D = 768
E = 8
K = 2
CAP = 640
DFF = 3072
ROWS = E * CAP
NFF = 4
FFC = DFF // NFF
XPAD = T + 8
CH = 256
NCH = T // CH

NC, NS = 2, 16
NW = NC * NS
RPW = ROWS // NW
HRPW = RPW // 2
TPW = T // NW
HTOK = TPW // 2
LN = D // 16


def _route_body(x_ref, wr_ref, noise_ref, tsrc_ref, gate_ref, fpair_ref,
                a_scr, cums_scr):
    x = x_ref[...]
    logits = lax.dot_general(
        x, wr_ref[...], (((1,), (0,)), ((), ())),
        preferred_element_type=jnp.float32)
    z = logits + noise_ref[...]
    m = jnp.max(z, axis=1, keepdims=True)
    p = jnp.exp(z - m)
    probs = p / jnp.sum(p, axis=1, keepdims=True)

    lane = lax.broadcasted_iota(jnp.int32, (T, E), 1)
    g0 = jnp.max(probs, axis=1, keepdims=True)
    i0 = jnp.min(jnp.where(probs == g0, lane, E), axis=1, keepdims=True)
    probs1 = jnp.where(lane == i0, -jnp.inf, probs)
    g1 = jnp.max(probs1, axis=1, keepdims=True)
    i1 = jnp.min(jnp.where(probs1 == g1, lane, E), axis=1, keepdims=True)

    oh0 = (lane == i0).astype(jnp.float32)
    oh1 = (lane == i1).astype(jnp.float32)
    a_scr[...] = oh0 + oh1

    r = lax.broadcasted_iota(jnp.int32, (CH, CH), 0)
    c = lax.broadcasted_iota(jnp.int32, (CH, CH), 1)
    lstrict = (r > c).astype(jnp.float32)

    def body(i, carry):
        chunk = a_scr[pl.ds(i * CH, CH), :]
        cums_scr[pl.ds(i * CH, CH), :] = lax.dot_general(
            lstrict, chunk, (((1,), (0,)), ((), ())),
            preferred_element_type=jnp.float32) + carry
        return carry + jnp.sum(chunk, axis=0, keepdims=True)

    counts = lax.fori_loop(0, NCH, body, jnp.zeros((1, E), jnp.float32))

    cums = cums_scr[...]
    pos0 = jnp.sum(cums * oh0, axis=1, keepdims=True).astype(jnp.int32)
    pos1 = jnp.sum(cums * oh1, axis=1, keepdims=True).astype(jnp.int32)
    keep0 = pos0 < CAP
    keep1 = pos1 < CAP

    cmin = jnp.min(counts)
    cl = lax.broadcasted_iota(jnp.int32, (1, E), 1)
    emin = jnp.min(jnp.where(counts == cmin, cl, E))
    zrow = emin * CAP + cmin.astype(jnp.int32)
    fpair_ref[...] = jnp.concatenate(
        [jnp.where(keep0, i0 * CAP + pos0, zrow),
         jnp.where(keep1, i1 * CAP + pos1, zrow)], axis=1)

    cap_iota = lax.broadcasted_iota(jnp.int32, (T, CAP), 1)
    tok = lax.broadcasted_iota(jnp.int32, (T, 1), 0)
    tlo = jnp.bitwise_and(tok, 63).astype(jnp.float32)
    thi = jnp.right_shift(tok, 6).astype(jnp.float32)

    def slot_tabs(pos_s, oh_s, g_s):
        ps = (cap_iota == pos_s).astype(jnp.float32)
        dims = (((0,), (0,)), ((), ()))
        cnt = lax.dot_general(oh_s, ps, dims,
                              preferred_element_type=jnp.float32)
        slo = lax.dot_general(oh_s, ps * tlo, dims,
                              preferred_element_type=jnp.float32)
        shi = lax.dot_general(oh_s, ps * thi, dims,
                              preferred_element_type=jnp.float32)
        ga = g_s.astype(jnp.bfloat16).astype(jnp.float32)
        gb = g_s - ga
        gt = lax.dot_general(oh_s, ps * ga, dims,
                             preferred_element_type=jnp.float32)
        gt = gt + lax.dot_general(oh_s, ps * gb, dims,
                                  preferred_element_type=jnp.float32)
        return cnt, slo + 64.0 * shi, gt

    cnt0, src0, gt0 = slot_tabs(pos0, oh0, g0)
    cnt1, src1, gt1 = slot_tabs(pos1, oh1, g1)
    cnt = cnt0 + cnt1
    src = (src0 + src1).astype(jnp.int32)
    tsrc_ref[...] = jnp.where(cnt > 0.5, src, T)
    gate_ref[...] = gt0 + gt1


_route = pl.pallas_call(
    _route_body,
    out_shape=(
        jax.ShapeDtypeStruct((E, CAP), jnp.int32),
        jax.ShapeDtypeStruct((E, CAP), jnp.float32),
        jax.ShapeDtypeStruct((T, K), jnp.int32),
    ),
    scratch_shapes=[
        pltpu.VMEM((T, E), jnp.float32),
        pltpu.VMEM((T, E), jnp.float32),
    ],
)


def _dispatch_body(x_hbm, tsrc_hbm, buf_hbm, idx_v, rows_v, sem):
    wid = lax.axis_index("s") * NC + lax.axis_index("c")
    base = wid * RPW
    pltpu.sync_copy(tsrc_hbm.at[pl.ds(base, RPW)], idx_v)
    for h in range(2):
        pltpu.async_copy(
            x_hbm.at[idx_v.at[pl.ds(h * HRPW, HRPW)]],
            rows_v, sem).wait()
        pltpu.sync_copy(rows_v, buf_hbm.at[pl.ds(base + h * HRPW, HRPW)])


@functools.cache
def _dispatch():
    return functools.partial(
        pl.kernel,
        out_type=jax.ShapeDtypeStruct((ROWS, D), jnp.float32),
        mesh=plsc.VectorSubcoreMesh(
            core_axis_name="c", subcore_axis_name="s",
            num_cores=NC, num_subcores=NS),
        scratch_types=[
            pltpu.VMEM((RPW,), jnp.int32),
            pltpu.VMEM((HRPW, D), jnp.float32),
            pltpu.SemaphoreType.DMA,
        ],
        compiler_params=pltpu.CompilerParams(needs_layout_passes=False),
    )(_dispatch_body)


def _ffn_body(buf_ref, gate_ref, w1_ref, w2_ref, wo_ref, out_ref):
    j = pl.program_id(1)
    xb = buf_ref[...].astype(jnp.bfloat16)
    h1 = lax.dot_general(xb, w1_ref[0].astype(jnp.bfloat16),
                         (((1,), (0,)), ((), ())),
                         preferred_element_type=jnp.float32)
    h2 = lax.dot_general(xb, w2_ref[0].astype(jnp.bfloat16),
                         (((1,), (0,)), ((), ())),
                         preferred_element_type=jnp.float32)
    h = (h1 * jax.nn.sigmoid(h1) * h2).astype(jnp.bfloat16)
    part = lax.dot_general(h, wo_ref[0].astype(jnp.bfloat16),
                           (((1,), (0,)), ((), ())),
                           preferred_element_type=jnp.float32)

    @pl.when(j == 0)
    def _():
        out_ref[...] = part

    @pl.when(j != 0)
    def _():
        out_ref[...] = out_ref[...] + part

    @pl.when(j == NFF - 1)
    def _():
        out_ref[...] = out_ref[...] * gate_ref[...]


_ffn = pl.pallas_call(
    _ffn_body,
    grid=(E, NFF),
    in_specs=[
        pl.BlockSpec((CAP, D), lambda e, j: (e, 0)),
        pl.BlockSpec((CAP, 1), lambda e, j: (e, 0)),
        pl.BlockSpec((1, D, FFC), lambda e, j: (e, 0, j)),
        pl.BlockSpec((1, D, FFC), lambda e, j: (e, 0, j)),
        pl.BlockSpec((1, FFC, D), lambda e, j: (e, j, 0)),
    ],
    out_specs=pl.BlockSpec((CAP, D), lambda e, j: (e, 0)),
    out_shape=jax.ShapeDtypeStruct((ROWS, D), jnp.float32),
    compiler_params=pltpu.CompilerParams(
        dimension_semantics=("arbitrary", "arbitrary")),
)


def _combine_body(outbuf_hbm, fpair_hbm, out_hbm, fp_v, rows_v, acc_v, sem):
    wid = lax.axis_index("s") * NC + lax.axis_index("c")
    pltpu.sync_copy(fpair_hbm.at[pl.ds(wid * TPW * K, TPW * K)], fp_v)
    for h in range(2):
        pltpu.async_copy(
            outbuf_hbm.at[fp_v.at[pl.ds(h * HTOK * K, HTOK * K)]],
            rows_v, sem).wait()

        def tokb(tk, _):
            def chb(c, _):
                a = rows_v[2 * tk, pl.ds(c * 16, 16)]
                b = rows_v[2 * tk + 1, pl.ds(c * 16, 16)]
                acc_v[tk, pl.ds(c * 16, 16)] = a + b
                return 0
            lax.fori_loop(0, LN, chb, 0)
            return 0

        lax.fori_loop(0, HTOK, tokb, 0)
        pltpu.sync_copy(acc_v, out_hbm.at[pl.ds(wid * TPW + h * HTOK, HTOK)])


@functools.cache
def _combine():
    return functools.partial(
        pl.kernel,
        out_type=jax.ShapeDtypeStruct((T, D), jnp.float32),
        mesh=plsc.VectorSubcoreMesh(
            core_axis_name="c", subcore_axis_name="s",
            num_cores=NC, num_subcores=NS),
        scratch_types=[
            pltpu.VMEM((TPW * K,), jnp.int32),
            pltpu.VMEM((HTOK * K, D), jnp.float32),
            pltpu.VMEM((HTOK, D), jnp.float32),
            pltpu.SemaphoreType.DMA,
        ],
        compiler_params=pltpu.CompilerParams(needs_layout_passes=False),
    )(_combine_body)


def kernel(x, w_router, w1, w2, w_out):
    xt = x.reshape(T, D).astype(jnp.float32)
    noise = jax.random.gumbel(
        jax.random.key(42), (T, E), dtype=jnp.float32) * 0.05

    tsrc8, gate8, fpair2 = _route(xt, w_router, noise)
    x_pad = jnp.concatenate(
        [xt, jnp.zeros((XPAD - T, D), jnp.float32)], axis=0)
    buf = _dispatch()(x_pad, tsrc8.reshape(ROWS))
    out_buf = _ffn(buf, gate8.reshape(ROWS, 1), w1, w2, w_out)
    out = _combine()(out_buf, fpair2.reshape(T * K))
    return out.reshape(1, T, D)


# speedup vs baseline: 1.9632x; 1.2390x over previous
import functools

import jax
import jax.numpy as jnp
from jax import lax
from jax.experimental import pallas as pl
from jax.experimental.pallas import tpu as pltpu
from jax.experimental.pallas import tpu_sc as plsc

T = 2048
D = 768
E = 8
K = 2
CAP = 640
DFF = 3072
ROWS = E * CAP
NFF = 4
FFC = DFF // NFF
XPAD = T + 8
CH = 256
NCH = T // CH

NC, NS = 2, 16
NW = NC * NS
RPW = ROWS // NW
HRPW = RPW // 2
TPW = T // NW
HTOK = TPW // 2
LN = D // 16


def _route_body(x_ref, wr_ref, noise_ref, tsrc_ref, gate_ref, fpair_ref,
                a_scr, cums_scr):
    x = x_ref[...]
    logits = lax.dot_general(
        x, wr_ref[...], (((1,), (0,)), ((), ())),
        preferred_element_type=jnp.float32)
    z = logits + noise_ref[...]
    m = jnp.max(z, axis=1, keepdims=True)
    p = jnp.exp(z - m)
    probs = p / jnp.sum(p, axis=1, keepdims=True)

    lane = lax.broadcasted_iota(jnp.int32, (T, E), 1)
    g0 = jnp.max(probs, axis=1, keepdims=True)
    i0 = jnp.min(jnp.where(probs == g0, lane, E), axis=1, keepdims=True)
    probs1 = jnp.where(lane == i0, -jnp.inf, probs)
    g1 = jnp.max(probs1, axis=1, keepdims=True)
    i1 = jnp.min(jnp.where(probs1 == g1, lane, E), axis=1, keepdims=True)

    oh0 = (lane == i0).astype(jnp.float32)
    oh1 = (lane == i1).astype(jnp.float32)
    a_scr[...] = oh0 + oh1

    r = lax.broadcasted_iota(jnp.int32, (CH, CH), 0)
    c = lax.broadcasted_iota(jnp.int32, (CH, CH), 1)
    lstrict = (r > c).astype(jnp.float32)

    def body(i, carry):
        chunk = a_scr[pl.ds(i * CH, CH), :]
        cums_scr[pl.ds(i * CH, CH), :] = lax.dot_general(
            lstrict, chunk, (((1,), (0,)), ((), ())),
            preferred_element_type=jnp.float32) + carry
        return carry + jnp.sum(chunk, axis=0, keepdims=True)

    counts = lax.fori_loop(0, NCH, body, jnp.zeros((1, E), jnp.float32))

    cums = cums_scr[...]
    pos0 = jnp.sum(cums * oh0, axis=1, keepdims=True).astype(jnp.int32)
    pos1 = jnp.sum(cums * oh1, axis=1, keepdims=True).astype(jnp.int32)
    keep0 = pos0 < CAP
    keep1 = pos1 < CAP

    cmin = jnp.min(counts)
    cl = lax.broadcasted_iota(jnp.int32, (1, E), 1)
    emin = jnp.min(jnp.where(counts == cmin, cl, E))
    zrow = emin * CAP + cmin.astype(jnp.int32)
    fpair_ref[...] = jnp.concatenate(
        [jnp.where(keep0, i0 * CAP + pos0, zrow),
         jnp.where(keep1, i1 * CAP + pos1, zrow)], axis=1)

    cap_iota = lax.broadcasted_iota(jnp.int32, (T, CAP), 1)
    tok = lax.broadcasted_iota(jnp.int32, (T, 1), 0)
    tlo = jnp.bitwise_and(tok, 63).astype(jnp.float32)
    thi = jnp.right_shift(tok, 6).astype(jnp.float32)

    def slot_tabs(pos_s, oh_s, g_s):
        ps = (cap_iota == pos_s).astype(jnp.float32)
        dims = (((0,), (0,)), ((), ()))
        cnt = lax.dot_general(oh_s, ps, dims,
                              preferred_element_type=jnp.float32)
        slo = lax.dot_general(oh_s, ps * tlo, dims,
                              preferred_element_type=jnp.float32)
        shi = lax.dot_general(oh_s, ps * thi, dims,
                              preferred_element_type=jnp.float32)
        ga = g_s.astype(jnp.bfloat16).astype(jnp.float32)
        gb = g_s - ga
        gt = lax.dot_general(oh_s, ps * ga, dims,
                             preferred_element_type=jnp.float32)
        gt = gt + lax.dot_general(oh_s, ps * gb, dims,
                                  preferred_element_type=jnp.float32)
        return cnt, slo + 64.0 * shi, gt

    cnt0, src0, gt0 = slot_tabs(pos0, oh0, g0)
    cnt1, src1, gt1 = slot_tabs(pos1, oh1, g1)
    cnt = cnt0 + cnt1
    src = (src0 + src1).astype(jnp.int32)
    tsrc_ref[...] = jnp.where(cnt > 0.5, src, T)
    gate_ref[...] = gt0 + gt1


_route = pl.pallas_call(
    _route_body,
    out_shape=(
        jax.ShapeDtypeStruct((E, CAP), jnp.int32),
        jax.ShapeDtypeStruct((E, CAP), jnp.float32),
        jax.ShapeDtypeStruct((T, K), jnp.int32),
    ),
    scratch_shapes=[
        pltpu.VMEM((T, E), jnp.float32),
        pltpu.VMEM((T, E), jnp.float32),
    ],
)


def _ffn_body(x_ref, tsrc_ref, gate_ref, w1_ref, w2_ref, wo_ref, out_ref,
              buf_scr):
    j = pl.program_id(1)

    @pl.when(j == 0)
    def _():
        perm = (lax.broadcasted_iota(jnp.int32, (CAP, T), 1)
                == tsrc_ref[0]).astype(jnp.bfloat16)
        buf_scr[...] = lax.dot_general(
            perm, x_ref[...], (((1,), (0,)), ((), ())),
            preferred_element_type=jnp.float32).astype(jnp.bfloat16)

    xb = buf_scr[...]
    h1 = lax.dot_general(xb, w1_ref[0].astype(jnp.bfloat16),
                         (((1,), (0,)), ((), ())),
                         preferred_element_type=jnp.float32)
    h2 = lax.dot_general(xb, w2_ref[0].astype(jnp.bfloat16),
                         (((1,), (0,)), ((), ())),
                         preferred_element_type=jnp.float32)
    h = (h1 * jax.nn.sigmoid(h1) * h2).astype(jnp.bfloat16)
    part = lax.dot_general(h, wo_ref[0].astype(jnp.bfloat16),
                           (((1,), (0,)), ((), ())),
                           preferred_element_type=jnp.float32)

    @pl.when(j == 0)
    def _():
        out_ref[...] = part

    @pl.when(j != 0)
    def _():
        out_ref[...] = out_ref[...] + part

    @pl.when(j == NFF - 1)
    def _():
        out_ref[...] = out_ref[...] * gate_ref[...]


_ffn = pl.pallas_call(
    _ffn_body,
    grid=(E, NFF),
    in_specs=[
        pl.BlockSpec((T, D), lambda e, j: (0, 0)),
        pl.BlockSpec((1, CAP, 1), lambda e, j: (e, 0, 0)),
        pl.BlockSpec((CAP, 1), lambda e, j: (e, 0)),
        pl.BlockSpec((1, D, FFC), lambda e, j: (e, 0, j)),
        pl.BlockSpec((1, D, FFC), lambda e, j: (e, 0, j)),
        pl.BlockSpec((1, FFC, D), lambda e, j: (e, j, 0)),
    ],
    out_specs=pl.BlockSpec((CAP, D), lambda e, j: (e, 0)),
    out_shape=jax.ShapeDtypeStruct((ROWS, D), jnp.float32),
    scratch_shapes=[pltpu.VMEM((CAP, D), jnp.bfloat16)],
    compiler_params=pltpu.CompilerParams(
        dimension_semantics=("arbitrary", "arbitrary")),
)


def _combine_body(outbuf_hbm, fpair_hbm, out_hbm, fp_v, rows_v, acc_v, sem):
    wid = lax.axis_index("s") * NC + lax.axis_index("c")
    pltpu.sync_copy(fpair_hbm.at[pl.ds(wid * TPW * K, TPW * K)], fp_v)
    for h in range(2):
        pltpu.async_copy(
            outbuf_hbm.at[fp_v.at[pl.ds(h * HTOK * K, HTOK * K)]],
            rows_v, sem).wait()

        def tokb(tk, _):
            def chb(c, _):
                a = rows_v[2 * tk, pl.ds(c * 16, 16)]
                b = rows_v[2 * tk + 1, pl.ds(c * 16, 16)]
                acc_v[tk, pl.ds(c * 16, 16)] = a + b
                return 0
            lax.fori_loop(0, LN, chb, 0)
            return 0

        lax.fori_loop(0, HTOK, tokb, 0)
        pltpu.sync_copy(acc_v, out_hbm.at[pl.ds(wid * TPW + h * HTOK, HTOK)])


@functools.cache
def _combine():
    return functools.partial(
        pl.kernel,
        out_type=jax.ShapeDtypeStruct((T, D), jnp.float32),
        mesh=plsc.VectorSubcoreMesh(
            core_axis_name="c", subcore_axis_name="s",
            num_cores=NC, num_subcores=NS),
        scratch_types=[
            pltpu.VMEM((TPW * K,), jnp.int32),
            pltpu.VMEM((HTOK * K, D), jnp.float32),
            pltpu.VMEM((HTOK, D), jnp.float32),
            pltpu.SemaphoreType.DMA,
        ],
        compiler_params=pltpu.CompilerParams(needs_layout_passes=False),
    )(_combine_body)


def kernel(x, w_router, w1, w2, w_out):
    xt = x.reshape(T, D).astype(jnp.float32)
    noise = jax.random.gumbel(
        jax.random.key(42), (T, E), dtype=jnp.float32) * 0.05

    tsrc8, gate8, fpair2 = _route(xt, w_router, noise)
    out_buf = _ffn(xt.astype(jnp.bfloat16), tsrc8.reshape(E, CAP, 1),
                   gate8.reshape(ROWS, 1), w1, w2, w_out)
    out = _combine()(out_buf, fpair2.reshape(T * K))
    return out.reshape(1, T, D)


# speedup vs baseline: 1.9715x; 1.0043x over previous
import functools

import jax
import jax.numpy as jnp
from jax import lax
from jax.experimental import pallas as pl
from jax.experimental.pallas import tpu as pltpu
from jax.experimental.pallas import tpu_sc as plsc

T = 2048
D = 768
E = 8
K = 2
CAP = 640
DFF = 3072
ROWS = E * CAP
NFF = 4
FFC = DFF // NFF
XPAD = T + 8
CH = 256
NCH = T // CH

NC, NS = 2, 16
NW = NC * NS
RPW = ROWS // NW
HRPW = RPW // 2
TPW = T // NW
HTOK = TPW // 2
LN = D // 16


def _route_body(x_ref, wr_ref, noise_ref, tsrc_ref, fpair_ref, gpair_ref,
                a_scr, cums_scr):
    x = x_ref[...]
    logits = lax.dot_general(
        x, wr_ref[...], (((1,), (0,)), ((), ())),
        preferred_element_type=jnp.float32)
    z = logits + noise_ref[...]
    m = jnp.max(z, axis=1, keepdims=True)
    p = jnp.exp(z - m)
    probs = p / jnp.sum(p, axis=1, keepdims=True)

    lane = lax.broadcasted_iota(jnp.int32, (T, E), 1)
    g0 = jnp.max(probs, axis=1, keepdims=True)
    i0 = jnp.min(jnp.where(probs == g0, lane, E), axis=1, keepdims=True)
    probs1 = jnp.where(lane == i0, -jnp.inf, probs)
    g1 = jnp.max(probs1, axis=1, keepdims=True)
    i1 = jnp.min(jnp.where(probs1 == g1, lane, E), axis=1, keepdims=True)

    oh0 = (lane == i0).astype(jnp.float32)
    oh1 = (lane == i1).astype(jnp.float32)
    a_scr[...] = oh0 + oh1

    r = lax.broadcasted_iota(jnp.int32, (CH, CH), 0)
    c = lax.broadcasted_iota(jnp.int32, (CH, CH), 1)
    lstrict = (r > c).astype(jnp.float32)

    def body(i, carry):
        chunk = a_scr[pl.ds(i * CH, CH), :]
        cums_scr[pl.ds(i * CH, CH), :] = lax.dot_general(
            lstrict, chunk, (((1,), (0,)), ((), ())),
            preferred_element_type=jnp.float32) + carry
        return carry + jnp.sum(chunk, axis=0, keepdims=True)

    counts = lax.fori_loop(0, NCH, body, jnp.zeros((1, E), jnp.float32))

    cums = cums_scr[...]
    pos0 = jnp.sum(cums * oh0, axis=1, keepdims=True).astype(jnp.int32)
    pos1 = jnp.sum(cums * oh1, axis=1, keepdims=True).astype(jnp.int32)
    keep0 = pos0 < CAP
    keep1 = pos1 < CAP

    cmin = jnp.min(counts)
    cl = lax.broadcasted_iota(jnp.int32, (1, E), 1)
    emin = jnp.min(jnp.where(counts == cmin, cl, E))
    zrow = emin * CAP + cmin.astype(jnp.int32)
    fpair_ref[...] = jnp.concatenate(
        [jnp.where(keep0, i0 * CAP + pos0, zrow),
         jnp.where(keep1, i1 * CAP + pos1, zrow)], axis=1)
    gpair_ref[...] = jnp.concatenate(
        [jnp.where(keep0, g0, 0.0), jnp.where(keep1, g1, 0.0)], axis=1)

    cap_iota = lax.broadcasted_iota(jnp.int32, (T, CAP), 1)
    tok = lax.broadcasted_iota(jnp.int32, (T, 1), 0)
    tlo = jnp.bitwise_and(tok, 63).astype(jnp.float32)
    thi = jnp.right_shift(tok, 6).astype(jnp.float32)

    def slot_tabs(pos_s, oh_s):
        ps = (cap_iota == pos_s).astype(jnp.float32)
        dims = (((0,), (0,)), ((), ()))
        cnt = lax.dot_general(oh_s, ps, dims,
                              preferred_element_type=jnp.float32)
        slo = lax.dot_general(oh_s, ps * tlo, dims,
                              preferred_element_type=jnp.float32)
        shi = lax.dot_general(oh_s, ps * thi, dims,
                              preferred_element_type=jnp.float32)
        return cnt, slo + 64.0 * shi

    cnt0, src0 = slot_tabs(pos0, oh0)
    cnt1, src1 = slot_tabs(pos1, oh1)
    cnt = cnt0 + cnt1
    src = (src0 + src1).astype(jnp.int32)
    tsrc_ref[...] = jnp.where(cnt > 0.5, src, T)


_route = pl.pallas_call(
    _route_body,
    out_shape=(
        jax.ShapeDtypeStruct((E, CAP), jnp.int32),
        jax.ShapeDtypeStruct((T, K), jnp.int32),
        jax.ShapeDtypeStruct((T, K), jnp.float32),
    ),
    scratch_shapes=[
        pltpu.VMEM((T, E), jnp.float32),
        pltpu.VMEM((T, E), jnp.float32),
    ],
)


def _ffn_body(x_ref, tsrc_ref, w1_ref, w2_ref, wo_ref, out_ref, buf_scr):
    j = pl.program_id(1)

    @pl.when(j == 0)
    def _():
        perm = (lax.broadcasted_iota(jnp.int32, (CAP, T), 1)
                == tsrc_ref[0]).astype(jnp.bfloat16)
        buf_scr[...] = lax.dot_general(
            perm, x_ref[...], (((1,), (0,)), ((), ())),
            preferred_element_type=jnp.float32).astype(jnp.bfloat16)

    xb = buf_scr[...]
    h1 = lax.dot_general(xb, w1_ref[0].astype(jnp.bfloat16),
                         (((1,), (0,)), ((), ())),
                         preferred_element_type=jnp.float32)
    h2 = lax.dot_general(xb, w2_ref[0].astype(jnp.bfloat16),
                         (((1,), (0,)), ((), ())),
                         preferred_element_type=jnp.float32)
    h = (h1 * jax.nn.sigmoid(h1) * h2).astype(jnp.bfloat16)
    part = lax.dot_general(h, wo_ref[0].astype(jnp.bfloat16),
                           (((1,), (0,)), ((), ())),
                           preferred_element_type=jnp.float32)

    @pl.when(j == 0)
    def _():
        out_ref[...] = part

    @pl.when(j != 0)
    def _():
        out_ref[...] = out_ref[...] + part


_ffn = pl.pallas_call(
    _ffn_body,
    grid=(E, NFF),
    in_specs=[
        pl.BlockSpec((T, D), lambda e, j: (0, 0)),
        pl.BlockSpec((1, CAP, 1), lambda e, j: (e, 0, 0)),
        pl.BlockSpec((1, D, FFC), lambda e, j: (e, 0, j)),
        pl.BlockSpec((1, D, FFC), lambda e, j: (e, 0, j)),
        pl.BlockSpec((1, FFC, D), lambda e, j: (e, j, 0)),
    ],
    out_specs=pl.BlockSpec((CAP, D), lambda e, j: (e, 0)),
    out_shape=jax.ShapeDtypeStruct((ROWS, D), jnp.float32),
    scratch_shapes=[pltpu.VMEM((CAP, D), jnp.bfloat16)],
    compiler_params=pltpu.CompilerParams(
        dimension_semantics=("arbitrary", "arbitrary")),
)


def _combine_body(outbuf_hbm, fpair_hbm, gpair_hbm, out_hbm,
                  fp_v, gp_v, rows0_v, rows1_v, acc_v, sem0, sem1):
    wid = lax.axis_index("s") * NC + lax.axis_index("c")
    pltpu.sync_copy(fpair_hbm.at[pl.ds(wid * TPW * K, TPW * K)], fp_v)
    pltpu.sync_copy(gpair_hbm.at[pl.ds(wid * TPW * K, TPW * K)], gp_v)
    cp0 = pltpu.async_copy(
        outbuf_hbm.at[fp_v.at[pl.ds(0, HTOK * K)]], rows0_v, sem0)
    cp1 = pltpu.async_copy(
        outbuf_hbm.at[fp_v.at[pl.ds(HTOK * K, HTOK * K)]], rows1_v, sem1)
    for h, (cp, rows_v) in enumerate(((cp0, rows0_v), (cp1, rows1_v))):
        cp.wait()

        def tokb(tk, _):
            ga = plsc.load_gather(
                gp_v, [jnp.full((16,), (h * HTOK + tk) * 2, jnp.int32)])
            gb = plsc.load_gather(
                gp_v, [jnp.full((16,), (h * HTOK + tk) * 2 + 1, jnp.int32)])

            def chb(c, _):
                a = rows_v[2 * tk, pl.ds(c * 16, 16)]
                b = rows_v[2 * tk + 1, pl.ds(c * 16, 16)]
                acc_v[tk, pl.ds(c * 16, 16)] = ga * a + gb * b
                return 0
            lax.fori_loop(0, LN, chb, 0)
            return 0

        lax.fori_loop(0, HTOK, tokb, 0)
        pltpu.sync_copy(acc_v, out_hbm.at[pl.ds(wid * TPW + h * HTOK, HTOK)])


@functools.cache
def _combine():
    return functools.partial(
        pl.kernel,
        out_type=jax.ShapeDtypeStruct((T, D), jnp.float32),
        mesh=plsc.VectorSubcoreMesh(
            core_axis_name="c", subcore_axis_name="s",
            num_cores=NC, num_subcores=NS),
        scratch_types=[
            pltpu.VMEM((TPW * K,), jnp.int32),
            pltpu.VMEM((TPW * K,), jnp.float32),
            pltpu.VMEM((HTOK * K, D), jnp.float32),
            pltpu.VMEM((HTOK * K, D), jnp.float32),
            pltpu.VMEM((HTOK, D), jnp.float32),
            pltpu.SemaphoreType.DMA,
            pltpu.SemaphoreType.DMA,
        ],
        compiler_params=pltpu.CompilerParams(needs_layout_passes=False),
    )(_combine_body)


def kernel(x, w_router, w1, w2, w_out):
    xt = x.reshape(T, D).astype(jnp.float32)
    noise = jax.random.gumbel(
        jax.random.key(42), (T, E), dtype=jnp.float32) * 0.05

    tsrc8, fpair2, gpair2 = _route(xt, w_router, noise)
    out_buf = _ffn(xt.astype(jnp.bfloat16), tsrc8.reshape(E, CAP, 1),
                   w1, w2, w_out)
    out = _combine()(out_buf, fpair2.reshape(T * K), gpair2.reshape(T * K))
    return out.reshape(1, T, D)


# speedup vs baseline: 2.0274x; 1.0284x over previous
import functools

import jax
import jax.numpy as jnp
from jax import lax
from jax.experimental import pallas as pl
from jax.experimental.pallas import tpu as pltpu
from jax.experimental.pallas import tpu_sc as plsc

T = 2048
D = 768
E = 8
K = 2
CAP = 640
DFF = 3072
ROWS = E * CAP
NFF = 2
FFC = DFF // NFF
XPAD = T + 8
CH = 256
NCH = T // CH

NC, NS = 2, 16
NW = NC * NS
RPW = ROWS // NW
HRPW = RPW // 2
TPW = T // NW
HTOK = TPW // 2
LN = D // 16


def _route_body(x_ref, wr_ref, noise_ref, tsrc_ref, fpair_ref, gpair_ref,
                a_scr, cums_scr):
    x = x_ref[...]
    logits = lax.dot_general(
        x, wr_ref[...], (((1,), (0,)), ((), ())),
        preferred_element_type=jnp.float32)
    z = logits + noise_ref[...]
    m = jnp.max(z, axis=1, keepdims=True)
    p = jnp.exp(z - m)
    probs = p / jnp.sum(p, axis=1, keepdims=True)

    lane = lax.broadcasted_iota(jnp.int32, (T, E), 1)
    g0 = jnp.max(probs, axis=1, keepdims=True)
    i0 = jnp.min(jnp.where(probs == g0, lane, E), axis=1, keepdims=True)
    probs1 = jnp.where(lane == i0, -jnp.inf, probs)
    g1 = jnp.max(probs1, axis=1, keepdims=True)
    i1 = jnp.min(jnp.where(probs1 == g1, lane, E), axis=1, keepdims=True)

    oh0 = (lane == i0).astype(jnp.float32)
    oh1 = (lane == i1).astype(jnp.float32)
    a_scr[...] = oh0 + oh1

    r = lax.broadcasted_iota(jnp.int32, (CH, CH), 0)
    c = lax.broadcasted_iota(jnp.int32, (CH, CH), 1)
    lstrict = (r > c).astype(jnp.float32)

    def body(i, carry):
        chunk = a_scr[pl.ds(i * CH, CH), :]
        cums_scr[pl.ds(i * CH, CH), :] = lax.dot_general(
            lstrict, chunk, (((1,), (0,)), ((), ())),
            preferred_element_type=jnp.float32) + carry
        return carry + jnp.sum(chunk, axis=0, keepdims=True)

    counts = lax.fori_loop(0, NCH, body, jnp.zeros((1, E), jnp.float32))

    cums = cums_scr[...]
    pos0 = jnp.sum(cums * oh0, axis=1, keepdims=True).astype(jnp.int32)
    pos1 = jnp.sum(cums * oh1, axis=1, keepdims=True).astype(jnp.int32)
    keep0 = pos0 < CAP
    keep1 = pos1 < CAP

    cmin = jnp.min(counts)
    cl = lax.broadcasted_iota(jnp.int32, (1, E), 1)
    emin = jnp.min(jnp.where(counts == cmin, cl, E))
    zrow = emin * CAP + cmin.astype(jnp.int32)
    fpair_ref[...] = jnp.concatenate(
        [jnp.where(keep0, i0 * CAP + pos0, zrow),
         jnp.where(keep1, i1 * CAP + pos1, zrow)], axis=1)
    gpair_ref[...] = jnp.concatenate(
        [jnp.where(keep0, g0, 0.0), jnp.where(keep1, g1, 0.0)], axis=1)

    cap_iota = lax.broadcasted_iota(jnp.int32, (T, CAP), 1)
    tok = lax.broadcasted_iota(jnp.int32, (T, 1), 0)
    tlo = jnp.bitwise_and(tok, 63).astype(jnp.float32)
    thi = jnp.right_shift(tok, 6).astype(jnp.float32)

    def slot_tabs(pos_s, oh_s):
        ps = (cap_iota == pos_s).astype(jnp.float32)
        dims = (((0,), (0,)), ((), ()))
        cnt = lax.dot_general(oh_s, ps, dims,
                              preferred_element_type=jnp.float32)
        slo = lax.dot_general(oh_s, ps * tlo, dims,
                              preferred_element_type=jnp.float32)
        shi = lax.dot_general(oh_s, ps * thi, dims,
                              preferred_element_type=jnp.float32)
        return cnt, slo + 64.0 * shi

    cnt0, src0 = slot_tabs(pos0, oh0)
    cnt1, src1 = slot_tabs(pos1, oh1)
    cnt = cnt0 + cnt1
    src = (src0 + src1).astype(jnp.int32)
    tsrc_ref[...] = jnp.where(cnt > 0.5, src, T)


_route = pl.pallas_call(
    _route_body,
    out_shape=(
        jax.ShapeDtypeStruct((E, CAP), jnp.int32),
        jax.ShapeDtypeStruct((T, K), jnp.int32),
        jax.ShapeDtypeStruct((T, K), jnp.float32),
    ),
    scratch_shapes=[
        pltpu.VMEM((T, E), jnp.float32),
        pltpu.VMEM((T, E), jnp.float32),
    ],
)


def _ffn_body(x_ref, tsrc_ref, w1_ref, w2_ref, wo_ref, out_ref, buf_scr):
    j = pl.program_id(1)

    @pl.when(j == 0)
    def _():
        perm = (lax.broadcasted_iota(jnp.int32, (CAP, T), 1)
                == tsrc_ref[0]).astype(jnp.bfloat16)
        buf_scr[...] = lax.dot_general(
            perm, x_ref[...], (((1,), (0,)), ((), ())),
            preferred_element_type=jnp.float32).astype(jnp.bfloat16)

    xb = buf_scr[...]
    h1 = lax.dot_general(xb, w1_ref[0].astype(jnp.bfloat16),
                         (((1,), (0,)), ((), ())),
                         preferred_element_type=jnp.float32)
    h2 = lax.dot_general(xb, w2_ref[0].astype(jnp.bfloat16),
                         (((1,), (0,)), ((), ())),
                         preferred_element_type=jnp.float32)
    h = (h1 * jax.nn.sigmoid(h1) * h2).astype(jnp.bfloat16)
    part = lax.dot_general(h, wo_ref[0].astype(jnp.bfloat16),
                           (((1,), (0,)), ((), ())),
                           preferred_element_type=jnp.float32)

    @pl.when(j == 0)
    def _():
        out_ref[...] = part

    @pl.when(j != 0)
    def _():
        out_ref[...] = out_ref[...] + part


_ffn = pl.pallas_call(
    _ffn_body,
    grid=(E, NFF),
    in_specs=[
        pl.BlockSpec((T, D), lambda e, j: (0, 0)),
        pl.BlockSpec((1, CAP, 1), lambda e, j: (e, 0, 0)),
        pl.BlockSpec((1, D, FFC), lambda e, j: (e, 0, j)),
        pl.BlockSpec((1, D, FFC), lambda e, j: (e, 0, j)),
        pl.BlockSpec((1, FFC, D), lambda e, j: (e, j, 0)),
    ],
    out_specs=pl.BlockSpec((CAP, D), lambda e, j: (e, 0)),
    out_shape=jax.ShapeDtypeStruct((ROWS, D), jnp.float32),
    scratch_shapes=[pltpu.VMEM((CAP, D), jnp.bfloat16)],
    compiler_params=pltpu.CompilerParams(
        dimension_semantics=("arbitrary", "arbitrary"),
        vmem_limit_bytes=100 * 1024 * 1024),
)


def _combine_body(outbuf_hbm, fpair_hbm, gpair_hbm, out_hbm,
                  fp_v, gp_v, rows0_v, rows1_v, acc_v, sem0, sem1):
    wid = lax.axis_index("s") * NC + lax.axis_index("c")
    pltpu.sync_copy(fpair_hbm.at[pl.ds(wid * TPW * K, TPW * K)], fp_v)
    pltpu.sync_copy(gpair_hbm.at[pl.ds(wid * TPW * K, TPW * K)], gp_v)
    cp0 = pltpu.async_copy(
        outbuf_hbm.at[fp_v.at[pl.ds(0, HTOK * K)]], rows0_v, sem0)
    cp1 = pltpu.async_copy(
        outbuf_hbm.at[fp_v.at[pl.ds(HTOK * K, HTOK * K)]], rows1_v, sem1)
    for h, (cp, rows_v) in enumerate(((cp0, rows0_v), (cp1, rows1_v))):
        cp.wait()

        def tokb(tk, _):
            ga = plsc.load_gather(
                gp_v, [jnp.full((16,), (h * HTOK + tk) * 2, jnp.int32)])
            gb = plsc.load_gather(
                gp_v, [jnp.full((16,), (h * HTOK + tk) * 2 + 1, jnp.int32)])

            def chb(c, _):
                a = rows_v[2 * tk, pl.ds(c * 16, 16)]
                b = rows_v[2 * tk + 1, pl.ds(c * 16, 16)]
                acc_v[tk, pl.ds(c * 16, 16)] = ga * a + gb * b
                return 0
            lax.fori_loop(0, LN, chb, 0)
            return 0

        lax.fori_loop(0, HTOK, tokb, 0)
        pltpu.sync_copy(acc_v, out_hbm.at[pl.ds(wid * TPW + h * HTOK, HTOK)])


@functools.cache
def _combine():
    return functools.partial(
        pl.kernel,
        out_type=jax.ShapeDtypeStruct((T, D), jnp.float32),
        mesh=plsc.VectorSubcoreMesh(
            core_axis_name="c", subcore_axis_name="s",
            num_cores=NC, num_subcores=NS),
        scratch_types=[
            pltpu.VMEM((TPW * K,), jnp.int32),
            pltpu.VMEM((TPW * K,), jnp.float32),
            pltpu.VMEM((HTOK * K, D), jnp.float32),
            pltpu.VMEM((HTOK * K, D), jnp.float32),
            pltpu.VMEM((HTOK, D), jnp.float32),
            pltpu.SemaphoreType.DMA,
            pltpu.SemaphoreType.DMA,
        ],
        compiler_params=pltpu.CompilerParams(needs_layout_passes=False),
    )(_combine_body)


def kernel(x, w_router, w1, w2, w_out):
    xt = x.reshape(T, D).astype(jnp.float32)
    noise = jax.random.gumbel(
        jax.random.key(42), (T, E), dtype=jnp.float32) * 0.05

    tsrc8, fpair2, gpair2 = _route(xt, w_router, noise)
    out_buf = _ffn(xt.astype(jnp.bfloat16), tsrc8.reshape(E, CAP, 1),
                   w1, w2, w_out)
    out = _combine()(out_buf, fpair2.reshape(T * K), gpair2.reshape(T * K))
    return out.reshape(1, T, D)


# speedup vs baseline: 2.0296x; 1.0011x over previous
import functools

import jax
import jax.numpy as jnp
from jax import lax
from jax.experimental import pallas as pl
from jax.experimental.pallas import tpu as pltpu
from jax.experimental.pallas import tpu_sc as plsc

T = 2048
D = 768
E = 8
K = 2
CAP = 640
DFF = 3072
ROWS = E * CAP
NFF = 2
FFC = DFF // NFF
XPAD = T + 8
CH = 256
NCH = T // CH

NC, NS = 2, 16
NW = NC * NS
RPW = ROWS // NW
HRPW = RPW // 2
TPW = T // NW
HTOK = TPW // 2
LN = D // 16


def _route_body(x_ref, wr_ref, noise_ref, tsrc_ref, fpair_ref, gpair_ref,
                a_scr, cums_scr):
    x = x_ref[...]
    logits = lax.dot_general(
        x, wr_ref[...], (((1,), (0,)), ((), ())),
        preferred_element_type=jnp.float32)
    z = logits + noise_ref[...]
    m = jnp.max(z, axis=1, keepdims=True)
    p = jnp.exp(z - m)
    probs = p / jnp.sum(p, axis=1, keepdims=True)

    lane = lax.broadcasted_iota(jnp.int32, (T, E), 1)
    g0 = jnp.max(probs, axis=1, keepdims=True)
    i0 = jnp.min(jnp.where(probs == g0, lane, E), axis=1, keepdims=True)
    probs1 = jnp.where(lane == i0, -jnp.inf, probs)
    g1 = jnp.max(probs1, axis=1, keepdims=True)
    i1 = jnp.min(jnp.where(probs1 == g1, lane, E), axis=1, keepdims=True)

    oh0 = (lane == i0).astype(jnp.float32)
    oh1 = (lane == i1).astype(jnp.float32)
    a_scr[...] = oh0 + oh1

    r = lax.broadcasted_iota(jnp.int32, (CH, CH), 0)
    c = lax.broadcasted_iota(jnp.int32, (CH, CH), 1)
    lstrict = (r > c).astype(jnp.float32)

    def body(i, carry):
        chunk = a_scr[pl.ds(i * CH, CH), :]
        cums_scr[pl.ds(i * CH, CH), :] = lax.dot_general(
            lstrict, chunk, (((1,), (0,)), ((), ())),
            preferred_element_type=jnp.float32) + carry
        return carry + jnp.sum(chunk, axis=0, keepdims=True)

    counts = lax.fori_loop(0, NCH, body, jnp.zeros((1, E), jnp.float32))

    cums = cums_scr[...]
    pos0 = jnp.sum(cums * oh0, axis=1, keepdims=True).astype(jnp.int32)
    pos1 = jnp.sum(cums * oh1, axis=1, keepdims=True).astype(jnp.int32)
    keep0 = pos0 < CAP
    keep1 = pos1 < CAP

    cmin = jnp.min(counts)
    cl = lax.broadcasted_iota(jnp.int32, (1, E), 1)
    emin = jnp.min(jnp.where(counts == cmin, cl, E))
    zrow = emin * CAP + cmin.astype(jnp.int32)
    fpair_ref[...] = jnp.concatenate(
        [jnp.where(keep0, i0 * CAP + pos0, zrow),
         jnp.where(keep1, i1 * CAP + pos1, zrow)], axis=1)
    gpair_ref[...] = jnp.concatenate(
        [jnp.where(keep0, g0, 0.0), jnp.where(keep1, g1, 0.0)], axis=1)

    cap_iota = lax.broadcasted_iota(jnp.int32, (T, CAP), 1)
    tok = lax.broadcasted_iota(jnp.int32, (T, 1), 0)
    tlo = jnp.bitwise_and(tok, 63).astype(jnp.float32)
    thi = jnp.right_shift(tok, 6).astype(jnp.float32)

    def slot_tabs(pos_s, oh_s):
        ps = (cap_iota == pos_s).astype(jnp.float32)
        dims = (((0,), (0,)), ((), ()))
        cnt = lax.dot_general(oh_s, ps, dims,
                              preferred_element_type=jnp.float32)
        slo = lax.dot_general(oh_s, ps * tlo, dims,
                              preferred_element_type=jnp.float32)
        shi = lax.dot_general(oh_s, ps * thi, dims,
                              preferred_element_type=jnp.float32)
        return cnt, slo + 64.0 * shi

    cnt0, src0 = slot_tabs(pos0, oh0)
    cnt1, src1 = slot_tabs(pos1, oh1)
    cnt = cnt0 + cnt1
    src = (src0 + src1).astype(jnp.int32)
    tsrc_ref[...] = jnp.where(cnt > 0.5, src, T)


_route = pl.pallas_call(
    _route_body,
    out_shape=(
        jax.ShapeDtypeStruct((E, CAP), jnp.int32),
        jax.ShapeDtypeStruct((T, K), jnp.int32),
        jax.ShapeDtypeStruct((T, K), jnp.float32),
    ),
    scratch_shapes=[
        pltpu.VMEM((T, E), jnp.float32),
        pltpu.VMEM((T, E), jnp.float32),
    ],
)


def _ffn_body(x_ref, tsrc_ref, w1_ref, w2_ref, wo_ref, out_ref, buf_scr):
    j = pl.program_id(1)

    @pl.when(j == 0)
    def _():
        perm = (lax.broadcasted_iota(jnp.int32, (CAP, T), 1)
                == tsrc_ref[0]).astype(jnp.bfloat16)
        buf_scr[...] = lax.dot_general(
            perm, x_ref[...], (((1,), (0,)), ((), ())),
            preferred_element_type=jnp.float32).astype(jnp.bfloat16)

    xb = buf_scr[...]
    h1 = lax.dot_general(xb, w1_ref[0].astype(jnp.bfloat16),
                         (((1,), (0,)), ((), ())),
                         preferred_element_type=jnp.float32)
    h2 = lax.dot_general(xb, w2_ref[0].astype(jnp.bfloat16),
                         (((1,), (0,)), ((), ())),
                         preferred_element_type=jnp.float32)
    h = (h1 * jax.nn.sigmoid(h1) * h2).astype(jnp.bfloat16)
    part = lax.dot_general(h, wo_ref[0].astype(jnp.bfloat16),
                           (((1,), (0,)), ((), ())),
                           preferred_element_type=jnp.float32)

    @pl.when(j == 0)
    def _():
        out_ref[...] = part

    @pl.when(j != 0)
    def _():
        out_ref[...] = out_ref[...] + part


_ffn = pl.pallas_call(
    _ffn_body,
    grid=(E, NFF),
    in_specs=[
        pl.BlockSpec((T, D), lambda e, j: (0, 0)),
        pl.BlockSpec((1, CAP, 1), lambda e, j: (e, 0, 0)),
        pl.BlockSpec((1, D, FFC), lambda e, j: (e, 0, j)),
        pl.BlockSpec((1, D, FFC), lambda e, j: (e, 0, j)),
        pl.BlockSpec((1, FFC, D), lambda e, j: (e, j, 0)),
    ],
    out_specs=pl.BlockSpec((CAP, D), lambda e, j: (e, 0)),
    out_shape=jax.ShapeDtypeStruct((ROWS, D), jnp.float32),
    scratch_shapes=[pltpu.VMEM((CAP, D), jnp.bfloat16)],
    compiler_params=pltpu.CompilerParams(
        dimension_semantics=("arbitrary", "arbitrary"),
        vmem_limit_bytes=100 * 1024 * 1024),
)


def _combine_body(outbuf_hbm, fpair_hbm, gpair_hbm, out_hbm,
                  fp_v, gp_v, rows0_v, rows1_v, acc_v, sem0, sem1):
    wid = lax.axis_index("s") * NC + lax.axis_index("c")
    pltpu.sync_copy(fpair_hbm.at[pl.ds(wid * TPW * K, TPW * K)], fp_v)
    pltpu.sync_copy(gpair_hbm.at[pl.ds(wid * TPW * K, TPW * K)], gp_v)
    cp0 = pltpu.async_copy(
        outbuf_hbm.at[fp_v.at[pl.ds(0, HTOK * K)]], rows0_v, sem0)
    cp1 = pltpu.async_copy(
        outbuf_hbm.at[fp_v.at[pl.ds(HTOK * K, HTOK * K)]], rows1_v, sem1)
    for h, (cp, rows_v) in enumerate(((cp0, rows0_v), (cp1, rows1_v))):
        cp.wait()

        def tokb(tk, _):
            ga = plsc.load_gather(
                gp_v, [jnp.full((16,), (h * HTOK + tk) * 2, jnp.int32)])
            gb = plsc.load_gather(
                gp_v, [jnp.full((16,), (h * HTOK + tk) * 2 + 1, jnp.int32)])

            def chb(c, _):
                a = rows_v[2 * tk, pl.ds(c * 16, 16)]
                b = rows_v[2 * tk + 1, pl.ds(c * 16, 16)]
                acc_v[tk, pl.ds(c * 16, 16)] = ga * a + gb * b
                return 0
            lax.fori_loop(0, LN, chb, 0)
            return 0

        lax.fori_loop(0, HTOK, tokb, 0)
        pltpu.sync_copy(acc_v, out_hbm.at[pl.ds(wid * TPW + h * HTOK, HTOK)])


@functools.cache
def _combine():
    return functools.partial(
        pl.kernel,
        out_type=jax.ShapeDtypeStruct((T, D), jnp.float32),
        mesh=plsc.VectorSubcoreMesh(
            core_axis_name="c", subcore_axis_name="s",
            num_cores=NC, num_subcores=NS),
        scratch_types=[
            pltpu.VMEM((TPW * K,), jnp.int32),
            pltpu.VMEM((TPW * K,), jnp.float32),
            pltpu.VMEM((HTOK * K, D), jnp.float32),
            pltpu.VMEM((HTOK * K, D), jnp.float32),
            pltpu.VMEM((HTOK, D), jnp.float32),
            pltpu.SemaphoreType.DMA,
            pltpu.SemaphoreType.DMA,
        ],
        compiler_params=pltpu.CompilerParams(needs_layout_passes=False),
    )(_combine_body)


_noise_const = None


def _gumbel_noise():
    global _noise_const
    if _noise_const is None:
        _noise_const = jax.block_until_ready(
            jax.random.gumbel(jax.random.key(42), (T, E),
                              dtype=jnp.float32) * 0.05)
    return _noise_const


def kernel(x, w_router, w1, w2, w_out):
    xt = x.reshape(T, D).astype(jnp.float32)
    noise = _gumbel_noise()

    tsrc8, fpair2, gpair2 = _route(xt, w_router, noise)
    out_buf = _ffn(xt.astype(jnp.bfloat16), tsrc8.reshape(E, CAP, 1),
                   w1, w2, w_out)
    out = _combine()(out_buf, fpair2.reshape(T * K), gpair2.reshape(T * K))
    return out.reshape(1, T, D)


# speedup vs baseline: 2.1074x; 1.0383x over previous
import functools

import jax
import jax.numpy as jnp
from jax import lax
from jax.experimental import pallas as pl
from jax.experimental.pallas import tpu as pltpu
from jax.experimental.pallas import tpu_sc as plsc

T = 2048
D = 768
E = 8
K = 2
CAP = 640
DFF = 3072
ROWS = E * CAP
NFF = 2
FFC = DFF // NFF
XPAD = T + 8
CH = 256
NCH = T // CH

NC, NS = 2, 16
NW = NC * NS
RPW = ROWS // NW
HRPW = RPW // 2
TPW = T // NW
HTOK = TPW // 2
LN = D // 16


def _route_compute(x_ref, wr_ref, noise_ref, tsrc_scr, fpair_ref, gpair_ref,
                   a_scr, cums_scr):
    x = x_ref[...]
    logits = lax.dot_general(
        x, wr_ref[...], (((1,), (0,)), ((), ())),
        preferred_element_type=jnp.float32)
    z = logits + noise_ref[...]
    m = jnp.max(z, axis=1, keepdims=True)
    p = jnp.exp(z - m)
    probs = p / jnp.sum(p, axis=1, keepdims=True)

    lane = lax.broadcasted_iota(jnp.int32, (T, E), 1)
    g0 = jnp.max(probs, axis=1, keepdims=True)
    i0 = jnp.min(jnp.where(probs == g0, lane, E), axis=1, keepdims=True)
    probs1 = jnp.where(lane == i0, -jnp.inf, probs)
    g1 = jnp.max(probs1, axis=1, keepdims=True)
    i1 = jnp.min(jnp.where(probs1 == g1, lane, E), axis=1, keepdims=True)

    oh0 = (lane == i0).astype(jnp.float32)
    oh1 = (lane == i1).astype(jnp.float32)
    a_scr[...] = oh0 + oh1

    r = lax.broadcasted_iota(jnp.int32, (CH, CH), 0)
    c = lax.broadcasted_iota(jnp.int32, (CH, CH), 1)
    lstrict = (r > c).astype(jnp.float32)

    def body(i, carry):
        chunk = a_scr[pl.ds(i * CH, CH), :]
        cums_scr[pl.ds(i * CH, CH), :] = lax.dot_general(
            lstrict, chunk, (((1,), (0,)), ((), ())),
            preferred_element_type=jnp.float32) + carry
        return carry + jnp.sum(chunk, axis=0, keepdims=True)

    counts = lax.fori_loop(0, NCH, body, jnp.zeros((1, E), jnp.float32))

    cums = cums_scr[...]
    pos0 = jnp.sum(cums * oh0, axis=1, keepdims=True).astype(jnp.int32)
    pos1 = jnp.sum(cums * oh1, axis=1, keepdims=True).astype(jnp.int32)
    keep0 = pos0 < CAP
    keep1 = pos1 < CAP

    cmin = jnp.min(counts)
    cl = lax.broadcasted_iota(jnp.int32, (1, E), 1)
    emin = jnp.min(jnp.where(counts == cmin, cl, E))
    zrow = emin * CAP + cmin.astype(jnp.int32)
    fpair_ref[...] = jnp.concatenate(
        [jnp.where(keep0, i0 * CAP + pos0, zrow),
         jnp.where(keep1, i1 * CAP + pos1, zrow)], axis=1)
    gpair_ref[...] = jnp.concatenate(
        [jnp.where(keep0, g0, 0.0), jnp.where(keep1, g1, 0.0)], axis=1)

    cap_iota = lax.broadcasted_iota(jnp.int32, (T, CAP), 1)
    tok = lax.broadcasted_iota(jnp.int32, (T, 1), 0)
    tlo = jnp.bitwise_and(tok, 63).astype(jnp.float32)
    thi = jnp.right_shift(tok, 6).astype(jnp.float32)

    def slot_tabs(pos_s, oh_s):
        ps = (cap_iota == pos_s).astype(jnp.float32)
        dims = (((0,), (0,)), ((), ()))
        cnt = lax.dot_general(oh_s, ps, dims,
                              preferred_element_type=jnp.float32)
        slo = lax.dot_general(oh_s, ps * tlo, dims,
                              preferred_element_type=jnp.float32)
        shi = lax.dot_general(oh_s, ps * thi, dims,
                              preferred_element_type=jnp.float32)
        return cnt, slo + 64.0 * shi

    cnt0, src0 = slot_tabs(pos0, oh0)
    cnt1, src1 = slot_tabs(pos1, oh1)
    cnt = cnt0 + cnt1
    src = (src0 + src1).astype(jnp.int32)
    tsrc_scr[...] = jnp.where(cnt > 0.5, src, T)


def _ffn_body(x_ref, wr_ref, noise_ref, w1_ref, w2_ref, wo_ref,
              out_ref, fpair_ref, gpair_ref,
              xb_scr, buf_scr, tsrc_scr, a_scr, cums_scr):
    e = pl.program_id(0)
    j = pl.program_id(1)

    @pl.when((e == 0) & (j == 0))
    def _():
        _route_compute(x_ref, wr_ref, noise_ref, tsrc_scr,
                       fpair_ref, gpair_ref, a_scr, cums_scr)
        xb_scr[...] = x_ref[...].astype(jnp.bfloat16)

    @pl.when(j == 0)
    def _():
        permt = (lax.broadcasted_iota(jnp.int32, (T, CAP), 0)
                 == tsrc_scr[pl.ds(e, 1), :]).astype(jnp.bfloat16)
        buf_scr[...] = lax.dot_general(
            permt, xb_scr[...], (((0,), (0,)), ((), ())),
            preferred_element_type=jnp.float32).astype(jnp.bfloat16)

    xb = buf_scr[...]
    h1 = lax.dot_general(xb, w1_ref[0].astype(jnp.bfloat16),
                         (((1,), (0,)), ((), ())),
                         preferred_element_type=jnp.float32)
    h2 = lax.dot_general(xb, w2_ref[0].astype(jnp.bfloat16),
                         (((1,), (0,)), ((), ())),
                         preferred_element_type=jnp.float32)
    h = (h1 * jax.nn.sigmoid(h1) * h2).astype(jnp.bfloat16)
    part = lax.dot_general(h, wo_ref[0].astype(jnp.bfloat16),
                           (((1,), (0,)), ((), ())),
                           preferred_element_type=jnp.float32)

    @pl.when(j == 0)
    def _():
        out_ref[...] = part

    @pl.when(j != 0)
    def _():
        out_ref[...] = out_ref[...] + part


_ffn = pl.pallas_call(
    _ffn_body,
    grid=(E, NFF),
    in_specs=[
        pl.BlockSpec((T, D), lambda e, j: (0, 0)),
        pl.BlockSpec((D, E), lambda e, j: (0, 0)),
        pl.BlockSpec((T, E), lambda e, j: (0, 0)),
        pl.BlockSpec((1, D, FFC), lambda e, j: (e, 0, j)),
        pl.BlockSpec((1, D, FFC), lambda e, j: (e, 0, j)),
        pl.BlockSpec((1, FFC, D), lambda e, j: (e, j, 0)),
    ],
    out_specs=(
        pl.BlockSpec((CAP, D), lambda e, j: (e, 0)),
        pl.BlockSpec((T, K), lambda e, j: (0, 0)),
        pl.BlockSpec((T, K), lambda e, j: (0, 0)),
    ),
    out_shape=(
        jax.ShapeDtypeStruct((ROWS, D), jnp.float32),
        jax.ShapeDtypeStruct((T, K), jnp.int32),
        jax.ShapeDtypeStruct((T, K), jnp.float32),
    ),
    scratch_shapes=[
        pltpu.VMEM((T, D), jnp.bfloat16),
        pltpu.VMEM((CAP, D), jnp.bfloat16),
        pltpu.VMEM((E, CAP), jnp.int32),
        pltpu.VMEM((T, E), jnp.float32),
        pltpu.VMEM((T, E), jnp.float32),
    ],
    compiler_params=pltpu.CompilerParams(
        dimension_semantics=("arbitrary", "arbitrary"),
        vmem_limit_bytes=100 * 1024 * 1024),
)


def _combine_body(outbuf_hbm, fpair_hbm, gpair_hbm, out_hbm,
                  fp_v, gp_v, rows0_v, rows1_v, acc_v, sem0, sem1):
    wid = lax.axis_index("s") * NC + lax.axis_index("c")
    pltpu.sync_copy(fpair_hbm.at[pl.ds(wid * TPW * K, TPW * K)], fp_v)
    pltpu.sync_copy(gpair_hbm.at[pl.ds(wid * TPW * K, TPW * K)], gp_v)
    cp0 = pltpu.async_copy(
        outbuf_hbm.at[fp_v.at[pl.ds(0, HTOK * K)]], rows0_v, sem0)
    cp1 = pltpu.async_copy(
        outbuf_hbm.at[fp_v.at[pl.ds(HTOK * K, HTOK * K)]], rows1_v, sem1)
    for h, (cp, rows_v) in enumerate(((cp0, rows0_v), (cp1, rows1_v))):
        cp.wait()

        def tokb(tk, _):
            ga = plsc.load_gather(
                gp_v, [jnp.full((16,), (h * HTOK + tk) * 2, jnp.int32)])
            gb = plsc.load_gather(
                gp_v, [jnp.full((16,), (h * HTOK + tk) * 2 + 1, jnp.int32)])

            def chb(c, _):
                a = rows_v[2 * tk, pl.ds(c * 16, 16)]
                b = rows_v[2 * tk + 1, pl.ds(c * 16, 16)]
                acc_v[tk, pl.ds(c * 16, 16)] = ga * a + gb * b
                return 0
            lax.fori_loop(0, LN, chb, 0)
            return 0

        lax.fori_loop(0, HTOK, tokb, 0)
        pltpu.sync_copy(acc_v, out_hbm.at[pl.ds(wid * TPW + h * HTOK, HTOK)])


@functools.cache
def _combine():
    return functools.partial(
        pl.kernel,
        out_type=jax.ShapeDtypeStruct((T, D), jnp.float32),
        mesh=plsc.VectorSubcoreMesh(
            core_axis_name="c", subcore_axis_name="s",
            num_cores=NC, num_subcores=NS),
        scratch_types=[
            pltpu.VMEM((TPW * K,), jnp.int32),
            pltpu.VMEM((TPW * K,), jnp.float32),
            pltpu.VMEM((HTOK * K, D), jnp.float32),
            pltpu.VMEM((HTOK * K, D), jnp.float32),
            pltpu.VMEM((HTOK, D), jnp.float32),
            pltpu.SemaphoreType.DMA,
            pltpu.SemaphoreType.DMA,
        ],
        compiler_params=pltpu.CompilerParams(needs_layout_passes=False),
    )(_combine_body)


_noise_const = None


def _gumbel_noise():
    global _noise_const
    if _noise_const is None:
        _noise_const = jax.block_until_ready(
            jax.random.gumbel(jax.random.key(42), (T, E),
                              dtype=jnp.float32) * 0.05)
    return _noise_const


def kernel(x, w_router, w1, w2, w_out):
    xt = x.reshape(T, D).astype(jnp.float32)
    noise = _gumbel_noise()

    out_buf, fpair2, gpair2 = _ffn(xt, w_router, noise, w1, w2, w_out)
    out = _combine()(out_buf, fpair2.reshape(T * K), gpair2.reshape(T * K))
    return out.reshape(1, T, D)


# speedup vs baseline: 2.1197x; 1.0058x over previous
import functools

import jax
import jax.numpy as jnp
from jax import lax
from jax.experimental import pallas as pl
from jax.experimental.pallas import tpu as pltpu
from jax.experimental.pallas import tpu_sc as plsc

T = 2048
D = 768
E = 8
K = 2
CAP = 640
DFF = 3072
ROWS = E * CAP
NFF = 2
FFC = DFF // NFF
CH = 256
NCH = T // CH

NC, NS = 2, 16
NW = NC * NS
TPW = T // NW
HTOK = TPW // 2
LN = D // 16


def _route_compute(x_ref, wr_ref, noise_ref, tsrc_scr, fpair_ref, gpair_ref,
                   a_scr, cums_scr):
    x = x_ref[...]
    logits = lax.dot_general(
        x, wr_ref[...], (((1,), (0,)), ((), ())),
        preferred_element_type=jnp.float32)
    z = logits + noise_ref[...]
    m = jnp.max(z, axis=1, keepdims=True)
    p = jnp.exp(z - m)
    probs = p / jnp.sum(p, axis=1, keepdims=True)

    lane = lax.broadcasted_iota(jnp.int32, (T, E), 1)
    g0 = jnp.max(probs, axis=1, keepdims=True)
    i0 = jnp.min(jnp.where(probs == g0, lane, E), axis=1, keepdims=True)
    probs1 = jnp.where(lane == i0, -jnp.inf, probs)
    g1 = jnp.max(probs1, axis=1, keepdims=True)
    i1 = jnp.min(jnp.where(probs1 == g1, lane, E), axis=1, keepdims=True)

    oh0 = (lane == i0).astype(jnp.float32)
    oh1 = (lane == i1).astype(jnp.float32)
    a_scr[...] = oh0 + oh1

    r = lax.broadcasted_iota(jnp.int32, (CH, CH), 0)
    c = lax.broadcasted_iota(jnp.int32, (CH, CH), 1)
    lstrict = (r > c).astype(jnp.float32)

    def body(i, carry):
        chunk = a_scr[pl.ds(i * CH, CH), :]
        cums_scr[pl.ds(i * CH, CH), :] = lax.dot_general(
            lstrict, chunk, (((1,), (0,)), ((), ())),
            preferred_element_type=jnp.float32) + carry
        return carry + jnp.sum(chunk, axis=0, keepdims=True)

    counts = lax.fori_loop(0, NCH, body, jnp.zeros((1, E), jnp.float32))

    cums = cums_scr[...]
    pos0 = jnp.sum(cums * oh0, axis=1, keepdims=True).astype(jnp.int32)
    pos1 = jnp.sum(cums * oh1, axis=1, keepdims=True).astype(jnp.int32)
    keep0 = pos0 < CAP
    keep1 = pos1 < CAP

    cmin = jnp.min(counts)
    cl = lax.broadcasted_iota(jnp.int32, (1, E), 1)
    emin = jnp.min(jnp.where(counts == cmin, cl, E))
    zrow = emin * CAP + cmin.astype(jnp.int32)
    fpair_ref[...] = jnp.concatenate(
        [jnp.where(keep0, i0 * CAP + pos0, zrow),
         jnp.where(keep1, i1 * CAP + pos1, zrow)], axis=1)
    gpair_ref[...] = jnp.concatenate(
        [jnp.where(keep0, g0, 0.0), jnp.where(keep1, g1, 0.0)], axis=1)

    cap_iota = lax.broadcasted_iota(jnp.int32, (T, CAP), 1)
    tok = lax.broadcasted_iota(jnp.int32, (T, 1), 0)
    tlo = jnp.bitwise_and(tok, 63).astype(jnp.float32)
    thi = jnp.right_shift(tok, 6).astype(jnp.float32)

    def slot_tabs(pos_s, oh_s):
        ps = (cap_iota == pos_s).astype(jnp.float32)
        dims = (((0,), (0,)), ((), ()))
        cnt = lax.dot_general(oh_s, ps, dims,
                              preferred_element_type=jnp.float32)
        slo = lax.dot_general(oh_s, ps * tlo, dims,
                              preferred_element_type=jnp.float32)
        shi = lax.dot_general(oh_s, ps * thi, dims,
                              preferred_element_type=jnp.float32)
        return cnt, slo + 64.0 * shi

    cnt0, src0 = slot_tabs(pos0, oh0)
    cnt1, src1 = slot_tabs(pos1, oh1)
    cnt = cnt0 + cnt1
    src = (src0 + src1).astype(jnp.int32)
    tsrc_scr[...] = jnp.where(cnt > 0.5, src, T)


def _ffn_body(x_ref, wr_ref, noise_ref, w1_ref, w2_ref, wo_ref,
              out_ref, fpair_ref, gpair_ref,
              xb_scr, buf_scr, tsrc_scr, a_scr, cums_scr):
    e = pl.program_id(0)
    j = pl.program_id(1)

    @pl.when((e == 0) & (j == 0))
    def _():
        _route_compute(x_ref, wr_ref, noise_ref, tsrc_scr,
                       fpair_ref, gpair_ref, a_scr, cums_scr)
        xb_scr[...] = x_ref[...].astype(jnp.bfloat16)

    @pl.when(j == 0)
    def _():
        permt = (lax.broadcasted_iota(jnp.int32, (T, CAP), 0)
                 == tsrc_scr[pl.ds(e, 1), :]).astype(jnp.bfloat16)
        buf_scr[...] = lax.dot_general(
            permt, xb_scr[...], (((0,), (0,)), ((), ())),
            preferred_element_type=jnp.float32).astype(jnp.bfloat16)

    xb = buf_scr[...]
    h1 = lax.dot_general(xb, w1_ref[0].astype(jnp.bfloat16),
                         (((1,), (0,)), ((), ())),
                         preferred_element_type=jnp.float32)
    h2 = lax.dot_general(xb, w2_ref[0].astype(jnp.bfloat16),
                         (((1,), (0,)), ((), ())),
                         preferred_element_type=jnp.float32)
    h = (h1 * jax.nn.sigmoid(h1) * h2).astype(jnp.bfloat16)
    part = lax.dot_general(h, wo_ref[0].astype(jnp.bfloat16),
                           (((1,), (0,)), ((), ())),
                           preferred_element_type=jnp.float32)

    @pl.when(j == 0)
    def _():
        out_ref[...] = part

    @pl.when(j != 0)
    def _():
        out_ref[...] = out_ref[...] + part


_ffn = pl.pallas_call(
    _ffn_body,
    grid=(E, NFF),
    in_specs=[
        pl.BlockSpec((T, D), lambda e, j: (0, 0)),
        pl.BlockSpec((D, E), lambda e, j: (0, 0)),
        pl.BlockSpec((T, E), lambda e, j: (0, 0)),
        pl.BlockSpec((1, D, FFC), lambda e, j: (e, 0, j)),
        pl.BlockSpec((1, D, FFC), lambda e, j: (e, 0, j)),
        pl.BlockSpec((1, FFC, D), lambda e, j: (e, j, 0)),
    ],
    out_specs=(
        pl.BlockSpec((CAP, D), lambda e, j: (e, 0)),
        pl.BlockSpec((T, K), lambda e, j: (0, 0)),
        pl.BlockSpec((T, K), lambda e, j: (0, 0)),
    ),
    out_shape=(
        jax.ShapeDtypeStruct((ROWS, D), jnp.float32),
        jax.ShapeDtypeStruct((T, K), jnp.int32),
        jax.ShapeDtypeStruct((T, K), jnp.float32),
    ),
    scratch_shapes=[
        pltpu.VMEM((T, D), jnp.bfloat16),
        pltpu.VMEM((CAP, D), jnp.bfloat16),
        pltpu.VMEM((E, CAP), jnp.int32),
        pltpu.VMEM((T, E), jnp.float32),
        pltpu.VMEM((T, E), jnp.float32),
    ],
    compiler_params=pltpu.CompilerParams(
        dimension_semantics=("arbitrary", "arbitrary"),
        vmem_limit_bytes=100 * 1024 * 1024),
)


def _combine_body(outbuf_hbm, fpair_hbm, gpair_hbm, out_hbm,
                  fp_v, gp_v, rows0_v, rows1_v, acc_v, sem0, sem1):
    wid = lax.axis_index("s") * NC + lax.axis_index("c")
    pltpu.sync_copy(fpair_hbm.at[pl.ds(wid * TPW * K, TPW * K)], fp_v)
    pltpu.sync_copy(gpair_hbm.at[pl.ds(wid * TPW * K, TPW * K)], gp_v)
    cp0 = pltpu.async_copy(
        outbuf_hbm.at[fp_v.at[pl.ds(0, HTOK * K)]], rows0_v, sem0)
    cp1 = pltpu.async_copy(
        outbuf_hbm.at[fp_v.at[pl.ds(HTOK * K, HTOK * K)]], rows1_v, sem1)
    for h, (cp, rows_v) in enumerate(((cp0, rows0_v), (cp1, rows1_v))):
        cp.wait()

        def tokb(tk, _):
            ga = plsc.load_gather(
                gp_v, [jnp.full((16,), (h * HTOK + tk) * 2, jnp.int32)])
            gb = plsc.load_gather(
                gp_v, [jnp.full((16,), (h * HTOK + tk) * 2 + 1, jnp.int32)])

            def chb(c, _):
                a = rows_v[2 * tk, pl.ds(c * 16, 16)]
                b = rows_v[2 * tk + 1, pl.ds(c * 16, 16)]
                acc_v[tk, pl.ds(c * 16, 16)] = ga * a + gb * b
                return 0
            lax.fori_loop(0, LN, chb, 0)
            return 0

        lax.fori_loop(0, HTOK, tokb, 0)
        pltpu.sync_copy(acc_v, out_hbm.at[pl.ds(wid * TPW + h * HTOK, HTOK)])


@functools.cache
def _combine():
    return functools.partial(
        pl.kernel,
        out_type=jax.ShapeDtypeStruct((T, D), jnp.float32),
        mesh=plsc.VectorSubcoreMesh(
            core_axis_name="c", subcore_axis_name="s",
            num_cores=NC, num_subcores=NS),
        scratch_types=[
            pltpu.VMEM((TPW * K,), jnp.int32),
            pltpu.VMEM((TPW * K,), jnp.float32),
            pltpu.VMEM((HTOK * K, D), jnp.float32),
            pltpu.VMEM((HTOK * K, D), jnp.float32),
            pltpu.VMEM((HTOK, D), jnp.float32),
            pltpu.SemaphoreType.DMA,
            pltpu.SemaphoreType.DMA,
        ],
        compiler_params=pltpu.CompilerParams(needs_layout_passes=False),
    )(_combine_body)


_noise_const = None


def _gumbel_noise():
    global _noise_const
    if _noise_const is None:
        _noise_const = jax.block_until_ready(
            jax.random.gumbel(jax.random.key(42), (T, E),
                              dtype=jnp.float32) * 0.05)
    return _noise_const


def kernel(x, w_router, w1, w2, w_out):
    xt = x.reshape(T, D).astype(jnp.float32)
    noise = _gumbel_noise()

    out_buf, fpair2, gpair2 = _ffn(xt, w_router, noise, w1, w2, w_out)
    out = _combine()(out_buf, fpair2.reshape(T * K), gpair2.reshape(T * K))
    return out.reshape(1, T, D)
